# Initial kernel scaffold; baseline (speedup 1.0000x reference)
import jax
import jax.numpy as jnp
from jax.experimental import pallas as pl


def kernel(in_feat, edge_index, W1, b1, W2, b2):
    raise NotImplementedError("write your pallas kernel here")


# speedup vs baseline: 10.6817x; 10.6817x over previous
import functools

import jax
import jax.numpy as jnp
from jax import lax
from jax.experimental import pallas as pl
from jax.experimental.pallas import tpu as pltpu
from jax.experimental.pallas import tpu_sc as plsc

NC = 2
NS = 16
L = 16
B = 64
NBC = 8

def _mesh():
    return plsc.VectorSubcoreMesh(core_axis_name="c", subcore_axis_name="s",
                                  num_cores=NC, num_subcores=NS)


def _degree_body(nch, np_, src_hbm, dst_hbm, out_hbm,
                 idx_s, idx_d, ones_v, zrow_v, acc_o, acc_i):
    c = lax.axis_index("c")
    s = lax.axis_index("s")
    npt = np_ // NS

    for i in range(B // L):
        ones_v[pl.ds(i * L, L)] = jnp.ones((L,), jnp.float32)
    for i in range(npt // L):
        zrow_v[pl.ds(i * L, L)] = jnp.zeros((L,), jnp.float32)

    pltpu.sync_copy(zrow_v, acc_o.at[pl.ds(s * npt, npt)])
    pltpu.sync_copy(zrow_v, acc_i.at[pl.ds(s * npt, npt)])
    plsc.subcore_barrier()

    def chunk(k, carry):
        pltpu.sync_copy(src_hbm.at[c, s, k], idx_s)
        pltpu.sync_copy(dst_hbm.at[c, s, k], idx_d)

        def body(j, carry2):
            pltpu.sync_copy(ones_v, acc_o.at[idx_s.at[j]], add=True)
            pltpu.sync_copy(ones_v, acc_i.at[idx_d.at[j]], add=True)
            return carry2

        return lax.fori_loop(0, NBC, body, carry)

    lax.fori_loop(0, nch, chunk, 0)
    plsc.subcore_barrier()

    pltpu.sync_copy(acc_o.at[pl.ds(s * npt, npt)],
                    out_hbm.at[c, 0, pl.ds(s * npt, npt)])
    pltpu.sync_copy(acc_i.at[pl.ds(s * npt, npt)],
                    out_hbm.at[c, 1, pl.ds(s * npt, npt)])


def _degree_call(src_r, dst_r, np_):
    nch = src_r.shape[2]
    body = functools.partial(_degree_body, nch, np_)
    return pl.kernel(
        body,
        out_type=jax.ShapeDtypeStruct((NC, 2, np_), jnp.float32),
        mesh=_mesh(),
        compiler_params=pltpu.CompilerParams(use_tc_tiling_on_sc=False),
        scratch_types=[
            pltpu.VMEM((NBC, B), jnp.int32),
            pltpu.VMEM((NBC, B), jnp.int32),
            pltpu.VMEM((B,), jnp.float32),
            pltpu.VMEM((np_ // NS,), jnp.float32),
            pltpu.VMEM_SHARED((np_,), jnp.float32),
            pltpu.VMEM_SHARED((np_,), jnp.float32),
        ],
    )(src_r, dst_r)


def _segsum_body(nch, np_, d, table_hbm, src_hbm, dst_hbm, out_hbm,
                 idx_s, idx_d, rows, acc, sems):
    c = lax.axis_index("c")
    s = lax.axis_index("s")
    npt = np_ // NS

    def zrow(i, carry):
        for k in range(d // L):
            rows[0, i, pl.ds(k * L, L)] = jnp.zeros((L,), jnp.float32)
        return carry

    lax.fori_loop(0, B, zrow, 0)
    for k in range(npt // B):
        pltpu.sync_copy(rows.at[0], acc.at[pl.ds(s * npt + k * B, B)])
    plsc.subcore_barrier()

    nhalf = NBC // 2

    def chunk(k, carry):
        pltpu.sync_copy(src_hbm.at[c, s, k], idx_s)
        pltpu.sync_copy(dst_hbm.at[c, s, k], idx_d)

        pltpu.async_copy(table_hbm.at[idx_s.at[0]], rows.at[0], sems.at[0])

        def body(jj, carry2):
            j0 = 2 * jj
            j1 = j0 + 1
            pltpu.async_copy(table_hbm.at[idx_s.at[j1]], rows.at[1],
                             sems.at[1])
            pltpu.make_async_copy(table_hbm.at[idx_s.at[j0]], rows.at[0],
                                  sems.at[0]).wait()
            pltpu.sync_copy(rows.at[0], acc.at[idx_d.at[j0]], add=True)

            @pl.when(jj + 1 < nhalf)
            def _():
                pltpu.async_copy(table_hbm.at[idx_s.at[j0 + 2]], rows.at[0],
                                 sems.at[0])

            pltpu.make_async_copy(table_hbm.at[idx_s.at[j1]], rows.at[1],
                                  sems.at[1]).wait()
            pltpu.sync_copy(rows.at[1], acc.at[idx_d.at[j1]], add=True)
            return carry2

        return lax.fori_loop(0, nhalf, body, carry)

    lax.fori_loop(0, nch, chunk, 0)
    plsc.subcore_barrier()

    for k in range(npt // B):
        pltpu.sync_copy(acc.at[pl.ds(s * npt + k * B, B)],
                        out_hbm.at[c, pl.ds(s * npt + k * B, B)])


def _segsum_call(table, src_r, dst_r, np_):
    nch = src_r.shape[2]
    d = table.shape[1]
    body = functools.partial(_segsum_body, nch, np_, d)
    return pl.kernel(
        body,
        out_type=jax.ShapeDtypeStruct((NC, np_, d), jnp.float32),
        mesh=_mesh(),
        compiler_params=pltpu.CompilerParams(use_tc_tiling_on_sc=False),
        scratch_types=[
            pltpu.VMEM((NBC, B), jnp.int32),
            pltpu.VMEM((NBC, B), jnp.int32),
            pltpu.VMEM((2, B, d), jnp.float32),
            pltpu.VMEM_SHARED((np_, d), jnp.float32),
            pltpu.SemaphoreType.DMA((2,)),
        ],
    )(table, src_r, dst_r)


def _rs_out(degs):
    return lax.rsqrt(jnp.maximum(degs[:, 0:1] + degs[:, 2:3], 1.0))


def _rs_in(degs):
    return lax.rsqrt(jnp.maximum(degs[:, 1:2] + degs[:, 3:4], 1.0))


def _l1_body(x_ref, w1_ref, degs_ref, out_ref):
    y = jnp.dot(x_ref[...], w1_ref[...], preferred_element_type=jnp.float32)
    out_ref[...] = y * _rs_out(degs_ref[...])


def _mid_body(p_ref, degs_ref, b1_ref, w2_ref, out_ref):
    degs = degs_ref[...]
    agg = (p_ref[0] + p_ref[1]) * _rs_in(degs)
    h = jnp.maximum(agg + b1_ref[...], 0.0)
    out_ref[...] = jnp.dot(h * _rs_out(degs), w2_ref[...],
                           preferred_element_type=jnp.float32)


def _out_body(p_ref, degs_ref, b2_ref, out_ref):
    out_ref[...] = ((p_ref[0] + p_ref[1]) * _rs_in(degs_ref[...])
                    + b2_ref[...])


def _tc_call(body, out_shape, *args):
    return pl.pallas_call(
        body, out_shape=jax.ShapeDtypeStruct(out_shape, jnp.float32))(*args)


def kernel(in_feat, edge_index, W1, b1, W2, b2):
    n = in_feat.shape[0]
    e = edge_index.shape[1]
    d_in = W1.shape[0]
    d_h = W1.shape[1]
    d_out = W2.shape[1]

    np_ = -(-(n + 1) // (NS * B)) * (NS * B)
    unit = NC * NS * B * NBC
    ep = -(-e // unit) * unit
    nch = ep // unit

    src = edge_index[0].astype(jnp.int32)
    dst = edge_index[1].astype(jnp.int32)
    fill = jnp.full((ep - e,), np_ - 1, jnp.int32)
    src_r = jnp.concatenate([src, fill]).reshape(NC, NS, nch, NBC, B)
    dst_r = jnp.concatenate([dst, fill]).reshape(NC, NS, nch, NBC, B)

    x_pad = jnp.pad(in_feat, ((0, np_ - n), (0, 0)))

    deg2 = _degree_call(src_r, dst_r, np_)
    degs_t = jnp.transpose(deg2.reshape(2 * NC, np_))

    xt1 = _tc_call(_l1_body, (np_, d_h), x_pad, W1, degs_t)
    p1 = _segsum_call(xt1, src_r, dst_r, np_)
    xt2 = _tc_call(_mid_body, (np_, d_out), p1, degs_t,
                   b1.reshape(1, d_h), W2)
    p2 = _segsum_call(xt2, src_r, dst_r, np_)
    out = _tc_call(_out_body, (np_, d_out), p2, degs_t, b2.reshape(1, d_out))
    return out[:n]


# speedup vs baseline: 11.5274x; 1.0792x over previous
import functools

import jax
import jax.numpy as jnp
from jax import lax
from jax.experimental import pallas as pl
from jax.experimental.pallas import tpu as pltpu
from jax.experimental.pallas import tpu_sc as plsc

NC = 2
NS = 16
L = 16
B = 128
NBC = 8

def _mesh():
    return plsc.VectorSubcoreMesh(core_axis_name="c", subcore_axis_name="s",
                                  num_cores=NC, num_subcores=NS)


def _degree_body(nch, np_, src_hbm, dst_hbm, out_hbm,
                 idx_s, idx_d, ones_v, zrow_v, acc_o, acc_i):
    c = lax.axis_index("c")
    s = lax.axis_index("s")
    npt = np_ // NS

    for i in range(B // L):
        ones_v[pl.ds(i * L, L)] = jnp.ones((L,), jnp.float32)
    for i in range(npt // L):
        zrow_v[pl.ds(i * L, L)] = jnp.zeros((L,), jnp.float32)

    pltpu.sync_copy(zrow_v, acc_o.at[pl.ds(s * npt, npt)])
    pltpu.sync_copy(zrow_v, acc_i.at[pl.ds(s * npt, npt)])
    plsc.subcore_barrier()

    def chunk(k, carry):
        pltpu.sync_copy(src_hbm.at[c, s, k], idx_s)
        pltpu.sync_copy(dst_hbm.at[c, s, k], idx_d)

        def body(j, carry2):
            pltpu.sync_copy(ones_v, acc_o.at[idx_s.at[j]], add=True)
            pltpu.sync_copy(ones_v, acc_i.at[idx_d.at[j]], add=True)
            return carry2

        return lax.fori_loop(0, NBC, body, carry)

    lax.fori_loop(0, nch, chunk, 0)
    plsc.subcore_barrier()

    pltpu.sync_copy(acc_o.at[pl.ds(s * npt, npt)],
                    out_hbm.at[c, 0, pl.ds(s * npt, npt)])
    pltpu.sync_copy(acc_i.at[pl.ds(s * npt, npt)],
                    out_hbm.at[c, 1, pl.ds(s * npt, npt)])


def _degree_call(src_r, dst_r, np_):
    nch = src_r.shape[2]
    body = functools.partial(_degree_body, nch, np_)
    return pl.kernel(
        body,
        out_type=jax.ShapeDtypeStruct((NC, 2, np_), jnp.float32),
        mesh=_mesh(),
        compiler_params=pltpu.CompilerParams(use_tc_tiling_on_sc=False),
        scratch_types=[
            pltpu.VMEM((NBC, B), jnp.int32),
            pltpu.VMEM((NBC, B), jnp.int32),
            pltpu.VMEM((B,), jnp.float32),
            pltpu.VMEM((np_ // NS,), jnp.float32),
            pltpu.VMEM_SHARED((np_,), jnp.float32),
            pltpu.VMEM_SHARED((np_,), jnp.float32),
        ],
    )(src_r, dst_r)


def _segsum_body(nch, np_, d, table_hbm, src_hbm, dst_hbm, out_hbm,
                 idx_s, idx_d, rows, acc, sems):
    c = lax.axis_index("c")
    s = lax.axis_index("s")
    npt = np_ // NS

    def zrow(i, carry):
        for k in range(d // L):
            rows[0, i, pl.ds(k * L, L)] = jnp.zeros((L,), jnp.float32)
        return carry

    lax.fori_loop(0, B, zrow, 0)
    for k in range(npt // B):
        pltpu.sync_copy(rows.at[0], acc.at[pl.ds(s * npt + k * B, B)])
    plsc.subcore_barrier()

    nhalf = NBC // 2

    def chunk(k, carry):
        pltpu.sync_copy(src_hbm.at[c, s, k], idx_s)
        pltpu.sync_copy(dst_hbm.at[c, s, k], idx_d)

        pltpu.async_copy(table_hbm.at[idx_s.at[0]], rows.at[0], sems.at[0])

        def body(jj, carry2):
            j0 = 2 * jj
            j1 = j0 + 1
            pltpu.async_copy(table_hbm.at[idx_s.at[j1]], rows.at[1],
                             sems.at[1])
            pltpu.make_async_copy(table_hbm.at[idx_s.at[j0]], rows.at[0],
                                  sems.at[0]).wait()
            pltpu.sync_copy(rows.at[0], acc.at[idx_d.at[j0]], add=True)

            @pl.when(jj + 1 < nhalf)
            def _():
                pltpu.async_copy(table_hbm.at[idx_s.at[j0 + 2]], rows.at[0],
                                 sems.at[0])

            pltpu.make_async_copy(table_hbm.at[idx_s.at[j1]], rows.at[1],
                                  sems.at[1]).wait()
            pltpu.sync_copy(rows.at[1], acc.at[idx_d.at[j1]], add=True)
            return carry2

        return lax.fori_loop(0, nhalf, body, carry)

    lax.fori_loop(0, nch, chunk, 0)
    plsc.subcore_barrier()

    for k in range(npt // B):
        pltpu.sync_copy(acc.at[pl.ds(s * npt + k * B, B)],
                        out_hbm.at[c, pl.ds(s * npt + k * B, B)])


def _segsum_call(table, src_r, dst_r, np_):
    nch = src_r.shape[2]
    d = table.shape[1]
    body = functools.partial(_segsum_body, nch, np_, d)
    return pl.kernel(
        body,
        out_type=jax.ShapeDtypeStruct((NC, np_, d), jnp.float32),
        mesh=_mesh(),
        compiler_params=pltpu.CompilerParams(use_tc_tiling_on_sc=False),
        scratch_types=[
            pltpu.VMEM((NBC, B), jnp.int32),
            pltpu.VMEM((NBC, B), jnp.int32),
            pltpu.VMEM((2, B, d), jnp.float32),
            pltpu.VMEM_SHARED((np_, d), jnp.float32),
            pltpu.SemaphoreType.DMA((2,)),
        ],
    )(table, src_r, dst_r)


def _rs_out(degs):
    return lax.rsqrt(jnp.maximum(degs[:, 0:1] + degs[:, 2:3], 1.0))


def _rs_in(degs):
    return lax.rsqrt(jnp.maximum(degs[:, 1:2] + degs[:, 3:4], 1.0))


def _l1_body(x_ref, w1_ref, degs_ref, out_ref):
    y = jnp.dot(x_ref[...], w1_ref[...], preferred_element_type=jnp.float32)
    out_ref[...] = y * _rs_out(degs_ref[...])


def _mid_body(p_ref, degs_ref, b1_ref, w2_ref, out_ref):
    degs = degs_ref[...]
    agg = (p_ref[0] + p_ref[1]) * _rs_in(degs)
    h = jnp.maximum(agg + b1_ref[...], 0.0)
    out_ref[...] = jnp.dot(h * _rs_out(degs), w2_ref[...],
                           preferred_element_type=jnp.float32)


def _out_body(p_ref, degs_ref, b2_ref, out_ref):
    out_ref[...] = ((p_ref[0] + p_ref[1]) * _rs_in(degs_ref[...])
                    + b2_ref[...])


def _tc_call(body, out_shape, *args):
    return pl.pallas_call(
        body, out_shape=jax.ShapeDtypeStruct(out_shape, jnp.float32))(*args)


def kernel(in_feat, edge_index, W1, b1, W2, b2):
    n = in_feat.shape[0]
    e = edge_index.shape[1]
    d_in = W1.shape[0]
    d_h = W1.shape[1]
    d_out = W2.shape[1]

    np_ = -(-(n + 1) // (NS * B)) * (NS * B)
    unit = NC * NS * B * NBC
    ep = -(-e // unit) * unit
    nch = ep // unit

    src = edge_index[0].astype(jnp.int32)
    dst = edge_index[1].astype(jnp.int32)
    fill = jnp.full((ep - e,), np_ - 1, jnp.int32)
    src_r = jnp.concatenate([src, fill]).reshape(NC, NS, nch, NBC, B)
    dst_r = jnp.concatenate([dst, fill]).reshape(NC, NS, nch, NBC, B)

    x_pad = jnp.pad(in_feat, ((0, np_ - n), (0, 0)))

    deg2 = _degree_call(src_r, dst_r, np_)
    degs_t = jnp.transpose(deg2.reshape(2 * NC, np_))

    xt1 = _tc_call(_l1_body, (np_, d_h), x_pad, W1, degs_t)
    p1 = _segsum_call(xt1, src_r, dst_r, np_)
    xt2 = _tc_call(_mid_body, (np_, d_out), p1, degs_t,
                   b1.reshape(1, d_h), W2)
    p2 = _segsum_call(xt2, src_r, dst_r, np_)
    out = _tc_call(_out_body, (np_, d_out), p2, degs_t, b2.reshape(1, d_out))
    return out[:n]


# speedup vs baseline: 13.8273x; 1.1995x over previous
import functools

import jax
import jax.numpy as jnp
from jax import lax
from jax.experimental import pallas as pl
from jax.experimental.pallas import tpu as pltpu
from jax.experimental.pallas import tpu_sc as plsc

NC = 2
NS = 16
L = 16
B = 128
NBC = 8

def _mesh():
    return plsc.VectorSubcoreMesh(core_axis_name="c", subcore_axis_name="s",
                                  num_cores=NC, num_subcores=NS)


def _degree_body(nch, np_, src_hbm, dst_hbm, out_hbm,
                 idx_s, idx_d, ones_v, zrow_v, acc_o, acc_i):
    c = lax.axis_index("c")
    s = lax.axis_index("s")
    npt = np_ // NS

    for i in range(B // L):
        ones_v[pl.ds(i * L, L)] = jnp.ones((L,), jnp.float32)
    for i in range(npt // L):
        zrow_v[pl.ds(i * L, L)] = jnp.zeros((L,), jnp.float32)

    pltpu.sync_copy(zrow_v, acc_o.at[pl.ds(s * npt, npt)])
    pltpu.sync_copy(zrow_v, acc_i.at[pl.ds(s * npt, npt)])
    plsc.subcore_barrier()

    def chunk(k, carry):
        pltpu.sync_copy(src_hbm.at[c, s, k], idx_s)
        pltpu.sync_copy(dst_hbm.at[c, s, k], idx_d)

        def body(j, carry2):
            pltpu.sync_copy(ones_v, acc_o.at[idx_s.at[j]], add=True)
            pltpu.sync_copy(ones_v, acc_i.at[idx_d.at[j]], add=True)
            return carry2

        return lax.fori_loop(0, NBC, body, carry)

    lax.fori_loop(0, nch, chunk, 0)
    plsc.subcore_barrier()

    pltpu.sync_copy(acc_o.at[pl.ds(s * npt, npt)],
                    out_hbm.at[c, 0, pl.ds(s * npt, npt)])
    pltpu.sync_copy(acc_i.at[pl.ds(s * npt, npt)],
                    out_hbm.at[c, 1, pl.ds(s * npt, npt)])


def _degree_call(src_r, dst_r, np_):
    nch = src_r.shape[2]
    body = functools.partial(_degree_body, nch, np_)
    return pl.kernel(
        body,
        out_type=jax.ShapeDtypeStruct((NC, 2, np_), jnp.float32),
        mesh=_mesh(),
        compiler_params=pltpu.CompilerParams(use_tc_tiling_on_sc=False),
        scratch_types=[
            pltpu.VMEM((NBC, B), jnp.int32),
            pltpu.VMEM((NBC, B), jnp.int32),
            pltpu.VMEM((B,), jnp.float32),
            pltpu.VMEM((np_ // NS,), jnp.float32),
            pltpu.VMEM_SHARED((np_,), jnp.float32),
            pltpu.VMEM_SHARED((np_,), jnp.float32),
        ],
    )(src_r, dst_r)


def _segsum_body(nch, np_, d, stage, table_hbm, src_hbm, dst_hbm, out_hbm,
                 *refs):
    if stage:
        table_sh, idx_s, idx_d, rows, acc, sems = refs
    else:
        idx_s, idx_d, rows, acc, sems = refs
        table_sh = table_hbm
    c = lax.axis_index("c")
    s = lax.axis_index("s")
    npt = np_ // NS

    if stage:
        pltpu.sync_copy(table_hbm.at[pl.ds(s * npt, npt)],
                        table_sh.at[pl.ds(s * npt, npt)])

    def zrow(i, carry):
        for k in range(d // L):
            rows[0, i, pl.ds(k * L, L)] = jnp.zeros((L,), jnp.float32)
        return carry

    lax.fori_loop(0, B, zrow, 0)
    for k in range(npt // B):
        pltpu.sync_copy(rows.at[0], acc.at[pl.ds(s * npt + k * B, B)])
    plsc.subcore_barrier()

    nhalf = NBC // 2

    def chunk(k, carry):
        pltpu.sync_copy(src_hbm.at[c, s, k], idx_s)
        pltpu.sync_copy(dst_hbm.at[c, s, k], idx_d)

        pltpu.async_copy(table_sh.at[idx_s.at[0]], rows.at[0], sems.at[0])

        def body(jj, carry2):
            j0 = 2 * jj
            j1 = j0 + 1
            pltpu.async_copy(table_sh.at[idx_s.at[j1]], rows.at[1],
                             sems.at[1])
            pltpu.make_async_copy(table_sh.at[idx_s.at[j0]], rows.at[0],
                                  sems.at[0]).wait()
            pltpu.sync_copy(rows.at[0], acc.at[idx_d.at[j0]], add=True)

            @pl.when(jj + 1 < nhalf)
            def _():
                pltpu.async_copy(table_sh.at[idx_s.at[j0 + 2]], rows.at[0],
                                 sems.at[0])

            pltpu.make_async_copy(table_sh.at[idx_s.at[j1]], rows.at[1],
                                  sems.at[1]).wait()
            pltpu.sync_copy(rows.at[1], acc.at[idx_d.at[j1]], add=True)
            return carry2

        return lax.fori_loop(0, nhalf, body, carry)

    lax.fori_loop(0, nch, chunk, 0)
    plsc.subcore_barrier()

    for k in range(npt // B):
        pltpu.sync_copy(acc.at[pl.ds(s * npt + k * B, B)],
                        out_hbm.at[c, pl.ds(s * npt + k * B, B)])


def _segsum_call(table, src_r, dst_r, np_, stage):
    nch = src_r.shape[2]
    d = table.shape[1]
    body = functools.partial(_segsum_body, nch, np_, d, stage)
    scratch = [
        pltpu.VMEM((NBC, B), jnp.int32),
        pltpu.VMEM((NBC, B), jnp.int32),
        pltpu.VMEM((2, B, d), jnp.float32),
        pltpu.VMEM_SHARED((np_, d), jnp.float32),
        pltpu.SemaphoreType.DMA((2,)),
    ]
    if stage:
        scratch.insert(0, pltpu.VMEM_SHARED((np_, d), jnp.float32))
    return pl.kernel(
        body,
        out_type=jax.ShapeDtypeStruct((NC, np_, d), jnp.float32),
        mesh=_mesh(),
        compiler_params=pltpu.CompilerParams(use_tc_tiling_on_sc=False),
        scratch_types=scratch,
    )(table, src_r, dst_r)


def _rs_out(degs):
    return lax.rsqrt(jnp.maximum(degs[:, 0:1] + degs[:, 2:3], 1.0))


def _rs_in(degs):
    return lax.rsqrt(jnp.maximum(degs[:, 1:2] + degs[:, 3:4], 1.0))


def _l1_body(x_ref, w1_ref, degs_ref, out_ref):
    y = jnp.dot(x_ref[...], w1_ref[...], preferred_element_type=jnp.float32)
    out_ref[...] = y * _rs_out(degs_ref[...])


def _mid_body(p_ref, degs_ref, b1_ref, w2_ref, out_ref):
    degs = degs_ref[...]
    agg = (p_ref[0] + p_ref[1]) * _rs_in(degs)
    h = jnp.maximum(agg + b1_ref[...], 0.0)
    out_ref[...] = jnp.dot(h * _rs_out(degs), w2_ref[...],
                           preferred_element_type=jnp.float32)


def _out_body(p_ref, degs_ref, b2_ref, out_ref):
    out_ref[...] = ((p_ref[0] + p_ref[1]) * _rs_in(degs_ref[...])
                    + b2_ref[...])


def _tc_call(body, out_shape, *args):
    return pl.pallas_call(
        body, out_shape=jax.ShapeDtypeStruct(out_shape, jnp.float32))(*args)


def kernel(in_feat, edge_index, W1, b1, W2, b2):
    n = in_feat.shape[0]
    e = edge_index.shape[1]
    d_in = W1.shape[0]
    d_h = W1.shape[1]
    d_out = W2.shape[1]

    np_ = -(-(n + 1) // (NS * B)) * (NS * B)
    unit = NC * NS * B * NBC
    ep = -(-e // unit) * unit
    nch = ep // unit

    src = edge_index[0].astype(jnp.int32)
    dst = edge_index[1].astype(jnp.int32)
    fill = jnp.full((ep - e,), np_ - 1, jnp.int32)
    src_r = jnp.concatenate([src, fill]).reshape(NC, NS, nch, NBC, B)
    dst_r = jnp.concatenate([dst, fill]).reshape(NC, NS, nch, NBC, B)

    x_pad = jnp.pad(in_feat, ((0, np_ - n), (0, 0)))

    deg2 = _degree_call(src_r, dst_r, np_)
    degs_t = jnp.transpose(deg2.reshape(2 * NC, np_))

    xt1 = _tc_call(_l1_body, (np_, d_h), x_pad, W1, degs_t)
    p1 = _segsum_call(xt1, src_r, dst_r, np_, stage=False)
    xt2 = _tc_call(_mid_body, (np_, d_out), p1, degs_t,
                   b1.reshape(1, d_h), W2)
    p2 = _segsum_call(xt2, src_r, dst_r, np_, stage=True)
    out = _tc_call(_out_body, (np_, d_out), p2, degs_t, b2.reshape(1, d_out))
    return out[:n]


# speedup vs baseline: 18.9176x; 1.3681x over previous
import functools

import jax
import jax.numpy as jnp
from jax import lax
from jax.experimental import pallas as pl
from jax.experimental.pallas import tpu as pltpu
from jax.experimental.pallas import tpu_sc as plsc

NC = 2
NS = 16
L = 16
B = 128
NBC = 8

def _mesh():
    return plsc.VectorSubcoreMesh(core_axis_name="c", subcore_axis_name="s",
                                  num_cores=NC, num_subcores=NS)


def _degree_body(nch, np_, src_hbm, dst_hbm, out_hbm,
                 idx_s, idx_d, ones_v, zrow_v, acc_o, acc_i):
    c = lax.axis_index("c")
    s = lax.axis_index("s")
    npt = np_ // NS

    for i in range(B // L):
        ones_v[pl.ds(i * L, L)] = jnp.ones((L,), jnp.float32)
    for i in range(npt // L):
        zrow_v[pl.ds(i * L, L)] = jnp.zeros((L,), jnp.float32)

    pltpu.sync_copy(zrow_v, acc_o.at[pl.ds(s * npt, npt)])
    pltpu.sync_copy(zrow_v, acc_i.at[pl.ds(s * npt, npt)])
    plsc.subcore_barrier()

    def chunk(k, carry):
        pltpu.sync_copy(src_hbm.at[c, s, k], idx_s)
        pltpu.sync_copy(dst_hbm.at[c, s, k], idx_d)

        def body(j, carry2):
            pltpu.sync_copy(ones_v, acc_o.at[idx_s.at[j]], add=True)
            pltpu.sync_copy(ones_v, acc_i.at[idx_d.at[j]], add=True)
            return carry2

        return lax.fori_loop(0, NBC, body, carry)

    lax.fori_loop(0, nch, chunk, 0)
    plsc.subcore_barrier()

    pltpu.sync_copy(acc_o.at[pl.ds(s * npt, npt)],
                    out_hbm.at[c, 0, pl.ds(s * npt, npt)])
    pltpu.sync_copy(acc_i.at[pl.ds(s * npt, npt)],
                    out_hbm.at[c, 1, pl.ds(s * npt, npt)])


def _degree_call(src_r, dst_r, np_):
    nch = src_r.shape[2]
    body = functools.partial(_degree_body, nch, np_)
    return pl.kernel(
        body,
        out_type=jax.ShapeDtypeStruct((NC, 2, np_), jnp.float32),
        mesh=_mesh(),
        compiler_params=pltpu.CompilerParams(use_tc_tiling_on_sc=False),
        scratch_types=[
            pltpu.VMEM((NBC, B), jnp.int32),
            pltpu.VMEM((NBC, B), jnp.int32),
            pltpu.VMEM((B,), jnp.float32),
            pltpu.VMEM((np_ // NS,), jnp.float32),
            pltpu.VMEM_SHARED((np_,), jnp.float32),
            pltpu.VMEM_SHARED((np_,), jnp.float32),
        ],
    )(src_r, dst_r)


def _segsum_body(nch, np_, d, stage, feat, table_hbm, src_hbm, dst_hbm,
                 out_hbm, *refs):
    if stage:
        table_sh, idx_s, idx_d, rows, acc, sems = refs
    else:
        idx_s, idx_d, rows, acc, sems = refs
        table_sh = table_hbm
    c = lax.axis_index("c")
    s = lax.axis_index("s")
    npt = np_ // NS

    if stage:
        tsrc = table_hbm.at[c] if feat else table_hbm
        pltpu.sync_copy(tsrc.at[pl.ds(s * npt, npt)],
                        table_sh.at[pl.ds(s * npt, npt)])

    def zrow(i, carry):
        for k in range(d // L):
            rows[0, i, pl.ds(k * L, L)] = jnp.zeros((L,), jnp.float32)
        return carry

    lax.fori_loop(0, B, zrow, 0)
    for k in range(npt // B):
        pltpu.sync_copy(rows.at[0], acc.at[pl.ds(s * npt + k * B, B)])
    plsc.subcore_barrier()

    nhalf = NBC // 2

    def chunk(k, carry):
        isrc = src_hbm.at[s, k] if feat else src_hbm.at[c, s, k]
        idst = dst_hbm.at[s, k] if feat else dst_hbm.at[c, s, k]
        pltpu.sync_copy(isrc, idx_s)
        pltpu.sync_copy(idst, idx_d)

        pltpu.async_copy(table_sh.at[idx_s.at[0]], rows.at[0], sems.at[0])

        def body(jj, carry2):
            j0 = 2 * jj
            j1 = j0 + 1
            pltpu.async_copy(table_sh.at[idx_s.at[j1]], rows.at[1],
                             sems.at[1])
            pltpu.make_async_copy(table_sh.at[idx_s.at[j0]], rows.at[0],
                                  sems.at[0]).wait()
            pltpu.sync_copy(rows.at[0], acc.at[idx_d.at[j0]], add=True)

            @pl.when(jj + 1 < nhalf)
            def _():
                pltpu.async_copy(table_sh.at[idx_s.at[j0 + 2]], rows.at[0],
                                 sems.at[0])

            pltpu.make_async_copy(table_sh.at[idx_s.at[j1]], rows.at[1],
                                  sems.at[1]).wait()
            pltpu.sync_copy(rows.at[1], acc.at[idx_d.at[j1]], add=True)
            return carry2

        return lax.fori_loop(0, nhalf, body, carry)

    lax.fori_loop(0, nch, chunk, 0)
    plsc.subcore_barrier()

    for k in range(npt // B):
        pltpu.sync_copy(acc.at[pl.ds(s * npt + k * B, B)],
                        out_hbm.at[c, pl.ds(s * npt + k * B, B)])


def _segsum_call(table, src_r, dst_r, np_, stage, feat=False):
    nch = src_r.shape[-3]
    d = table.shape[-1]
    body = functools.partial(_segsum_body, nch, np_, d, stage, feat)
    scratch = [
        pltpu.VMEM((NBC, B), jnp.int32),
        pltpu.VMEM((NBC, B), jnp.int32),
        pltpu.VMEM((2, B, d), jnp.float32),
        pltpu.VMEM_SHARED((np_, d), jnp.float32),
        pltpu.SemaphoreType.DMA((2,)),
    ]
    if stage:
        scratch.insert(0, pltpu.VMEM_SHARED((np_, d), jnp.float32))
    return pl.kernel(
        body,
        out_type=jax.ShapeDtypeStruct((NC, np_, d), jnp.float32),
        mesh=_mesh(),
        compiler_params=pltpu.CompilerParams(use_tc_tiling_on_sc=False),
        scratch_types=scratch,
    )(table, src_r, dst_r)


def _rs_out(degs):
    return lax.rsqrt(jnp.maximum(degs[:, 0:1] + degs[:, 2:3], 1.0))


def _rs_in(degs):
    return lax.rsqrt(jnp.maximum(degs[:, 1:2] + degs[:, 3:4], 1.0))


def _l1_body(x_ref, w1_ref, degs_ref, out_ref):
    y = jnp.dot(x_ref[...], w1_ref[...], preferred_element_type=jnp.float32)
    out_ref[...] = y * _rs_out(degs_ref[...])


def _mid_body(s1_ref, degs_ref, b1_ref, w2_ref, out_ref):
    degs = degs_ref[...]
    agg = s1_ref[...] * _rs_in(degs)
    h = jnp.maximum(agg + b1_ref[...], 0.0)
    out_ref[...] = jnp.dot(h * _rs_out(degs), w2_ref[...],
                           preferred_element_type=jnp.float32)


def _out_body(p_ref, degs_ref, b2_ref, out_ref):
    out_ref[...] = ((p_ref[0] + p_ref[1]) * _rs_in(degs_ref[...])
                    + b2_ref[...])


def _tc_call(body, out_shape, *args):
    return pl.pallas_call(
        body, out_shape=jax.ShapeDtypeStruct(out_shape, jnp.float32))(*args)


def kernel(in_feat, edge_index, W1, b1, W2, b2):
    n = in_feat.shape[0]
    e = edge_index.shape[1]
    d_in = W1.shape[0]
    d_h = W1.shape[1]
    d_out = W2.shape[1]

    np_ = -(-(n + 1) // (NS * B)) * (NS * B)
    unit = NC * NS * B * NBC
    ep = -(-e // unit) * unit
    nch = ep // unit

    src = edge_index[0].astype(jnp.int32)
    dst = edge_index[1].astype(jnp.int32)
    fill = jnp.full((ep - e,), np_ - 1, jnp.int32)
    src_flat = jnp.concatenate([src, fill])
    dst_flat = jnp.concatenate([dst, fill])
    src_r = src_flat.reshape(NC, NS, nch, NBC, B)
    dst_r = dst_flat.reshape(NC, NS, nch, NBC, B)
    src_f = src_flat.reshape(NS, NC * nch, NBC, B)
    dst_f = dst_flat.reshape(NS, NC * nch, NBC, B)

    x_pad = jnp.pad(in_feat, ((0, np_ - n), (0, 0)))

    deg2 = _degree_call(src_r, dst_r, np_)
    degs_t = jnp.transpose(deg2.reshape(2 * NC, np_))

    xt1 = _tc_call(_l1_body, (np_, d_h), x_pad, W1, degs_t)
    dhh = d_h // NC
    xt1_sp = xt1.reshape(np_, NC, dhh).transpose(1, 0, 2)
    p1 = _segsum_call(xt1_sp, src_f, dst_f, np_, stage=True, feat=True)
    s1 = p1.transpose(1, 0, 2).reshape(np_, d_h)
    xt2 = _tc_call(_mid_body, (np_, d_out), s1, degs_t,
                   b1.reshape(1, d_h), W2)
    p2 = _segsum_call(xt2, src_r, dst_r, np_, stage=True)
    out = _tc_call(_out_body, (np_, d_out), p2, degs_t, b2.reshape(1, d_out))
    return out[:n]


# speedup vs baseline: 22.5987x; 1.1946x over previous
import functools

import jax
import jax.numpy as jnp
from jax import lax
from jax.experimental import pallas as pl
from jax.experimental.pallas import tpu as pltpu
from jax.experimental.pallas import tpu_sc as plsc

NC = 2
NS = 16
L = 16
B = 128
NBC = 8

def _mesh():
    return plsc.VectorSubcoreMesh(core_axis_name="c", subcore_axis_name="s",
                                  num_cores=NC, num_subcores=NS)


def _degree_body(nch, np_, src_hbm, dst_hbm, out_hbm,
                 idx_s, idx_d, ones_v, zrow_v, acc_o, acc_i):
    c = lax.axis_index("c")
    s = lax.axis_index("s")
    npt = np_ // NS

    for i in range(B // L):
        ones_v[pl.ds(i * L, L)] = jnp.ones((L,), jnp.float32)
    for i in range(npt // L):
        zrow_v[pl.ds(i * L, L)] = jnp.zeros((L,), jnp.float32)

    pltpu.sync_copy(zrow_v, acc_o.at[pl.ds(s * npt, npt)])
    pltpu.sync_copy(zrow_v, acc_i.at[pl.ds(s * npt, npt)])
    plsc.subcore_barrier()

    def chunk(k, carry):
        pltpu.sync_copy(src_hbm.at[c, s, k], idx_s)
        pltpu.sync_copy(dst_hbm.at[c, s, k], idx_d)

        def body(j, carry2):
            pltpu.sync_copy(ones_v, acc_o.at[idx_s.at[j]], add=True)
            pltpu.sync_copy(ones_v, acc_i.at[idx_d.at[j]], add=True)
            return carry2

        return lax.fori_loop(0, NBC, body, carry)

    lax.fori_loop(0, nch, chunk, 0)
    plsc.subcore_barrier()

    pltpu.sync_copy(acc_o.at[pl.ds(s * npt, npt)],
                    out_hbm.at[c, 0, pl.ds(s * npt, npt)])
    pltpu.sync_copy(acc_i.at[pl.ds(s * npt, npt)],
                    out_hbm.at[c, 1, pl.ds(s * npt, npt)])


def _degree_call(src_r, dst_r, np_):
    nch = src_r.shape[2]
    body = functools.partial(_degree_body, nch, np_)
    return pl.kernel(
        body,
        out_type=jax.ShapeDtypeStruct((NC, 2, np_), jnp.float32),
        mesh=_mesh(),
        compiler_params=pltpu.CompilerParams(use_tc_tiling_on_sc=False),
        scratch_types=[
            pltpu.VMEM((NBC, B), jnp.int32),
            pltpu.VMEM((NBC, B), jnp.int32),
            pltpu.VMEM((B,), jnp.float32),
            pltpu.VMEM((np_ // NS,), jnp.float32),
            pltpu.VMEM_SHARED((np_,), jnp.float32),
            pltpu.VMEM_SHARED((np_,), jnp.float32),
        ],
    )(src_r, dst_r)


def _segsum_body(nch, np_, d, stage, feat, table_hbm, src_hbm, dst_hbm,
                 out_hbm, *refs):
    if stage:
        table_sh, idx_s, idx_d, rows, acc, gsem, ssem = refs
    else:
        idx_s, idx_d, rows, acc, gsem, ssem = refs
        table_sh = table_hbm
    c = lax.axis_index("c")
    s = lax.axis_index("s")
    npt = np_ // NS

    if stage:
        tsrc = table_hbm.at[c] if feat else table_hbm
        pltpu.sync_copy(tsrc.at[pl.ds(s * npt, npt)],
                        table_sh.at[pl.ds(s * npt, npt)])

    def zrow(i, carry):
        for k in range(d // L):
            rows[0, i, pl.ds(k * L, L)] = jnp.zeros((L,), jnp.float32)
        return carry

    lax.fori_loop(0, B, zrow, 0)
    for k in range(npt // B):
        pltpu.sync_copy(rows.at[0], acc.at[pl.ds(s * npt + k * B, B)])
    plsc.subcore_barrier()

    nb = nch * NBC

    def jbody(j, carry):
        k = j // NBC
        jj = j - k * NBC
        kb = lax.rem(k, 2)
        bg = lax.rem(j, 3)

        @pl.when(j >= 3)
        def _():
            jr = j - 3
            kr = jr // NBC
            jjr = jr - kr * NBC
            kbr = lax.rem(kr, 2)
            pltpu.make_async_copy(rows.at[bg], acc.at[idx_d.at[kbr, jjr]],
                                  ssem.at[bg]).wait()

        @pl.when(jj == 0)
        def _():
            isrc = src_hbm.at[s, k] if feat else src_hbm.at[c, s, k]
            idst = dst_hbm.at[s, k] if feat else dst_hbm.at[c, s, k]
            pltpu.sync_copy(isrc, idx_s.at[kb])
            pltpu.sync_copy(idst, idx_d.at[kb])

        pltpu.async_copy(table_sh.at[idx_s.at[kb, jj]], rows.at[bg],
                         gsem.at[bg])

        @pl.when(j >= 1)
        def _():
            jp = j - 1
            kp = jp // NBC
            jjp = jp - kp * NBC
            kbp = lax.rem(kp, 2)
            bs = lax.rem(jp, 3)
            pltpu.make_async_copy(table_sh.at[idx_s.at[kbp, jjp]],
                                  rows.at[bs], gsem.at[bs]).wait()
            pltpu.async_copy(rows.at[bs], acc.at[idx_d.at[kbp, jjp]],
                             ssem.at[bs], add=True)

        return carry

    lax.fori_loop(0, nb, jbody, 0)

    jp = nb - 1
    kp = jp // NBC
    jjp = jp - kp * NBC
    kbp = kp % 2
    bs = jp % 3
    pltpu.make_async_copy(table_sh.at[idx_s.at[kbp, jjp]],
                          rows.at[bs], gsem.at[bs]).wait()
    pltpu.async_copy(rows.at[bs], acc.at[idx_d.at[kbp, jjp]],
                     ssem.at[bs], add=True)
    for m in (nb - 1, nb - 2, nb - 3):
        km = m // NBC
        pltpu.make_async_copy(rows.at[m % 3],
                              acc.at[idx_d.at[km % 2, m - km * NBC]],
                              ssem.at[m % 3]).wait()

    plsc.subcore_barrier()

    for k in range(npt // B):
        pltpu.sync_copy(acc.at[pl.ds(s * npt + k * B, B)],
                        out_hbm.at[c, pl.ds(s * npt + k * B, B)])


def _segsum_call(table, src_r, dst_r, np_, stage, feat=False):
    nch = src_r.shape[-3]
    d = table.shape[-1]
    body = functools.partial(_segsum_body, nch, np_, d, stage, feat)
    scratch = [
        pltpu.VMEM((2, NBC, B), jnp.int32),
        pltpu.VMEM((2, NBC, B), jnp.int32),
        pltpu.VMEM((3, B, d), jnp.float32),
        pltpu.VMEM_SHARED((np_, d), jnp.float32),
        pltpu.SemaphoreType.DMA((3,)),
        pltpu.SemaphoreType.DMA((3,)),
    ]
    if stage:
        scratch.insert(0, pltpu.VMEM_SHARED((np_, d), jnp.float32))
    return pl.kernel(
        body,
        out_type=jax.ShapeDtypeStruct((NC, np_, d), jnp.float32),
        mesh=_mesh(),
        compiler_params=pltpu.CompilerParams(use_tc_tiling_on_sc=False),
        scratch_types=scratch,
    )(table, src_r, dst_r)


def _rs_out(degs):
    return lax.rsqrt(jnp.maximum(degs[:, 0:1] + degs[:, 2:3], 1.0))


def _rs_in(degs):
    return lax.rsqrt(jnp.maximum(degs[:, 1:2] + degs[:, 3:4], 1.0))


def _l1_body(x_ref, w1_ref, degs_ref, out_ref):
    y = jnp.dot(x_ref[...], w1_ref[...], preferred_element_type=jnp.float32)
    out_ref[...] = y * _rs_out(degs_ref[...])


def _mid_body(s1_ref, degs_ref, b1_ref, w2_ref, out_ref):
    degs = degs_ref[...]
    agg = s1_ref[...] * _rs_in(degs)
    h = jnp.maximum(agg + b1_ref[...], 0.0)
    out_ref[...] = jnp.dot(h * _rs_out(degs), w2_ref[...],
                           preferred_element_type=jnp.float32)


def _out_body(p_ref, degs_ref, b2_ref, out_ref):
    out_ref[...] = ((p_ref[0] + p_ref[1]) * _rs_in(degs_ref[...])
                    + b2_ref[...])


def _tc_call(body, out_shape, *args):
    return pl.pallas_call(
        body, out_shape=jax.ShapeDtypeStruct(out_shape, jnp.float32))(*args)


def kernel(in_feat, edge_index, W1, b1, W2, b2):
    n = in_feat.shape[0]
    e = edge_index.shape[1]
    d_in = W1.shape[0]
    d_h = W1.shape[1]
    d_out = W2.shape[1]

    np_ = -(-(n + 1) // (NS * B)) * (NS * B)
    unit = NC * NS * B * NBC
    ep = -(-e // unit) * unit
    nch = ep // unit

    src = edge_index[0].astype(jnp.int32)
    dst = edge_index[1].astype(jnp.int32)
    fill = jnp.full((ep - e,), np_ - 1, jnp.int32)
    src_flat = jnp.concatenate([src, fill])
    dst_flat = jnp.concatenate([dst, fill])
    src_r = src_flat.reshape(NC, NS, nch, NBC, B)
    dst_r = dst_flat.reshape(NC, NS, nch, NBC, B)
    src_f = src_flat.reshape(NS, NC * nch, NBC, B)
    dst_f = dst_flat.reshape(NS, NC * nch, NBC, B)

    x_pad = jnp.pad(in_feat, ((0, np_ - n), (0, 0)))

    deg2 = _degree_call(src_r, dst_r, np_)
    degs_t = jnp.transpose(deg2.reshape(2 * NC, np_))

    xt1 = _tc_call(_l1_body, (np_, d_h), x_pad, W1, degs_t)
    dhh = d_h // NC
    xt1_sp = xt1.reshape(np_, NC, dhh).transpose(1, 0, 2)
    p1 = _segsum_call(xt1_sp, src_f, dst_f, np_, stage=True, feat=True)
    s1 = p1.transpose(1, 0, 2).reshape(np_, d_h)
    xt2 = _tc_call(_mid_body, (np_, d_out), s1, degs_t,
                   b1.reshape(1, d_h), W2)
    p2 = _segsum_call(xt2, src_r, dst_r, np_, stage=True)
    out = _tc_call(_out_body, (np_, d_out), p2, degs_t, b2.reshape(1, d_out))
    return out[:n]
